# Initial kernel scaffold; baseline (speedup 1.0000x reference)
import jax
import jax.numpy as jnp
from jax.experimental import pallas as pl


def kernel(values_l0, values_l1, values_l2, values_l3):
    raise NotImplementedError("write your pallas kernel here")


# speedup vs baseline: 2.3280x; 2.3280x over previous
import functools
import math

import jax
import jax.numpy as jnp
from jax import lax
from jax.experimental import pallas as pl
from jax.experimental.pallas import tpu as pltpu
from jax.experimental.pallas import tpu_sc as plsc

N_SAMPLES = 20000
N_Q = 16
L_MAX = 3
N_OUT = 4 * N_Q * N_Q

NUM_CORES = 2
NUM_SUBCORES = 16
NUM_WORKERS = NUM_CORES * NUM_SUBCORES
PER_WORKER = N_SAMPLES // NUM_WORKERS
CHUNK = 25
NCHUNKS = PER_WORKER // CHUNK


def _body(v0_hbm, v1_hbm, v2_hbm, v3_hbm, out_hbm, b0, b1, b2, b3, ob):
    wid = lax.axis_index("s") * NUM_CORES + lax.axis_index("c")
    in_bufs = (b0, b1, b2, b3)
    in_hbm = (v0_hbm, v1_hbm, v2_hbm, v3_hbm)

    def chunk_body(c, _):
        base = wid * PER_WORKER + c * CHUNK
        for l in range(L_MAX + 1):
            pltpu.sync_copy(in_hbm[l].at[pl.ds(base, CHUNK)], in_bufs[l])

        def sample_body(s, _):
            for l in range(L_MAX + 1):
                ref = in_bufs[l]
                cg = 1.0 / math.sqrt(2 * l + 1)
                rows = [None] * N_Q
                for m in range(2 * l + 1):
                    vm = ref[s, m, :]
                    vms = vm * cg if l > 0 else vm
                    for q in range(N_Q):
                        t = vm[q] * vms
                        rows[q] = t if m == 0 else rows[q] + t
                col0 = l * N_Q * N_Q
                for q in range(N_Q):
                    ob[s, pl.ds(col0 + q * N_Q, N_Q)] = rows[q]
            return 0

        lax.fori_loop(0, CHUNK, sample_body, 0)
        pltpu.sync_copy(ob, out_hbm.at[pl.ds(base, CHUNK)])
        return 0

    lax.fori_loop(0, NCHUNKS, chunk_body, 0)


@jax.jit
def kernel(values_l0, values_l1, values_l2, values_l3):
    mesh = plsc.VectorSubcoreMesh(
        core_axis_name="c",
        subcore_axis_name="s",
        num_cores=NUM_CORES,
        num_subcores=NUM_SUBCORES,
    )
    scratch = [
        pltpu.VMEM((CHUNK, 2 * l + 1, N_Q), jnp.float32) for l in range(L_MAX + 1)
    ] + [pltpu.VMEM((CHUNK, N_OUT), jnp.float32)]
    run = pl.kernel(
        _body,
        out_type=jax.ShapeDtypeStruct((N_SAMPLES, N_OUT), jnp.float32),
        mesh=mesh,
        scratch_types=scratch,
        compiler_params=pltpu.CompilerParams(use_tc_tiling_on_sc=False),
    )
    return run(values_l0, values_l1, values_l2, values_l3)


# speedup vs baseline: 2.8343x; 1.2175x over previous
import functools
import math

import jax
import jax.numpy as jnp
from jax import lax
from jax.experimental import pallas as pl
from jax.experimental.pallas import tpu as pltpu
from jax.experimental.pallas import tpu_sc as plsc

N_SAMPLES = 20000
N_Q = 16
L_MAX = 3
N_OUT = 4 * N_Q * N_Q

NUM_CORES = 2
NUM_SUBCORES = 16
NUM_WORKERS = NUM_CORES * NUM_SUBCORES
PER_WORKER = N_SAMPLES // NUM_WORKERS
CHUNK = 25
NCHUNKS = PER_WORKER // CHUNK


def _compute_chunk(in_bufs, ob):
    def sample_body(s, _):
        for l in range(L_MAX + 1):
            ref = in_bufs[l]
            cg = 1.0 / math.sqrt(2 * l + 1)
            rows = [None] * N_Q
            for m in range(2 * l + 1):
                vm = ref[s, m, :]
                vms = vm * cg if l > 0 else vm
                for q in range(N_Q):
                    t = vm[q] * vms
                    rows[q] = t if m == 0 else rows[q] + t
            col0 = l * N_Q * N_Q
            for q in range(N_Q):
                ob[s, pl.ds(col0 + q * N_Q, N_Q)] = rows[q]
        return 0

    lax.fori_loop(0, CHUNK, sample_body, 0)


def _body(v0_hbm, v1_hbm, v2_hbm, v3_hbm, out_hbm,
          a0, a1, a2, a3, oa, b0, b1, b2, b3, obuf,
          sia, sib, soa, sob):
    wid = lax.axis_index("s") * NUM_CORES + lax.axis_index("c")
    in_hbm = (v0_hbm, v1_hbm, v2_hbm, v3_hbm)
    sets = (
        ((a0, a1, a2, a3), oa, sia, soa),
        ((b0, b1, b2, b3), obuf, sib, sob),
    )
    w0 = wid * PER_WORKER

    def issue_in(bufs, sem, base):
        for l in range(L_MAX + 1):
            pltpu.async_copy(in_hbm[l].at[pl.ds(base, CHUNK)], bufs[l], sem)

    def wait_in(bufs, sem):
        for l in range(L_MAX + 1):
            pltpu.make_async_copy(
                in_hbm[l].at[pl.ds(w0, CHUNK)], bufs[l], sem).wait()

    issue_in(sets[0][0], sets[0][2], w0)

    def chunk_body(c, _):
        base = w0 + c * CHUNK

        def do(par):
            bufs, ob, si, so = sets[par]
            nbufs, _, nsi, _ = sets[1 - par]
            wait_in(bufs, si)

            @pl.when(c + 1 < NCHUNKS)
            def _():
                issue_in(nbufs, nsi, base + CHUNK)

            @pl.when(c >= 2)
            def _():
                pltpu.make_async_copy(
                    ob, out_hbm.at[pl.ds(w0, CHUNK)], so).wait()

            _compute_chunk(bufs, ob)
            pltpu.async_copy(ob, out_hbm.at[pl.ds(base, CHUNK)], so)

        @pl.when(c % 2 == 0)
        def _():
            do(0)

        @pl.when(c % 2 == 1)
        def _():
            do(1)

        return 0

    lax.fori_loop(0, NCHUNKS, chunk_body, 0)
    pltpu.make_async_copy(sets[(NCHUNKS - 2) % 2][1],
                          out_hbm.at[pl.ds(w0, CHUNK)],
                          sets[(NCHUNKS - 2) % 2][3]).wait()
    pltpu.make_async_copy(sets[(NCHUNKS - 1) % 2][1],
                          out_hbm.at[pl.ds(w0, CHUNK)],
                          sets[(NCHUNKS - 1) % 2][3]).wait()


@jax.jit
def kernel(values_l0, values_l1, values_l2, values_l3):
    mesh = plsc.VectorSubcoreMesh(
        core_axis_name="c",
        subcore_axis_name="s",
        num_cores=NUM_CORES,
        num_subcores=NUM_SUBCORES,
    )
    scratch = (
        [pltpu.VMEM((CHUNK, 2 * l + 1, N_Q), jnp.float32) for l in range(L_MAX + 1)]
        + [pltpu.VMEM((CHUNK, N_OUT), jnp.float32)]
        + [pltpu.VMEM((CHUNK, 2 * l + 1, N_Q), jnp.float32) for l in range(L_MAX + 1)]
        + [pltpu.VMEM((CHUNK, N_OUT), jnp.float32)]
        + [pltpu.SemaphoreType.DMA] * 4
    )
    run = pl.kernel(
        _body,
        out_type=jax.ShapeDtypeStruct((N_SAMPLES, N_OUT), jnp.float32),
        mesh=mesh,
        scratch_types=scratch,
        compiler_params=pltpu.CompilerParams(use_tc_tiling_on_sc=False),
    )
    return run(values_l0, values_l1, values_l2, values_l3)


# speedup vs baseline: 4.6218x; 1.6307x over previous
import math

import jax
import jax.numpy as jnp
from jax import lax
from jax.experimental import pallas as pl
from jax.experimental.pallas import tpu as pltpu
from jax.experimental.pallas import tpu_sc as plsc

N_SAMPLES = 20000
N_Q = 16
L_MAX = 3
N_IN = 4 * 4 * N_Q
N_OUT = 4 * N_Q * N_Q
OFF = (0, 16, 64, 144)

NUM_CORES = 2
NUM_SUBCORES = 16
NUM_WORKERS = NUM_CORES * NUM_SUBCORES
CHUNK = 40
NCHUNKS_TOTAL = N_SAMPLES // CHUNK
NITER = -(-NCHUNKS_TOTAL // NUM_WORKERS)


def _compute_chunk(ib, ob):
    def sample_body(s, _):
        for l in range(L_MAX + 1):
            cg = 1.0 / math.sqrt(2 * l + 1)
            rows = [None] * N_Q
            for m in range(2 * l + 1):
                vm = ib[s, pl.ds(OFF[l] + m * N_Q, N_Q)]
                vms = vm * cg if l > 0 else vm
                for q in range(N_Q):
                    t = vm[q] * vms
                    rows[q] = t if m == 0 else rows[q] + t
            col0 = l * N_Q * N_Q
            for q in range(N_Q):
                ob[s, pl.ds(col0 + q * N_Q, N_Q)] = rows[q]
        return 0

    lax.fori_loop(0, CHUNK, sample_body, 0)


def _body(vin_hbm, out_hbm, ia, oa, ib, obuf, sia, sib, soa, sob):
    wid = lax.axis_index("s") * NUM_CORES + lax.axis_index("c")
    sets = ((ia, oa, sia, soa), (ib, obuf, sib, sob))

    pltpu.async_copy(vin_hbm.at[pl.ds(wid * CHUNK, CHUNK)], sets[0][0], sets[0][2])

    def iter_body(j, _):
        c = wid + j * NUM_WORKERS

        @pl.when(c < NCHUNKS_TOTAL)
        def _():
            def do(par):
                ibuf, ob, si, so = sets[par]
                nibuf, _, nsi, _ = sets[1 - par]
                base = c * CHUNK
                pltpu.make_async_copy(
                    vin_hbm.at[pl.ds(0, CHUNK)], ibuf, si).wait()

                @pl.when(c + NUM_WORKERS < NCHUNKS_TOTAL)
                def _():
                    pltpu.async_copy(
                        vin_hbm.at[pl.ds(base + NUM_WORKERS * CHUNK, CHUNK)],
                        nibuf, nsi)

                @pl.when(j >= 2)
                def _():
                    pltpu.make_async_copy(
                        ob, out_hbm.at[pl.ds(0, CHUNK)], so).wait()

                _compute_chunk(ibuf, ob)
                pltpu.async_copy(ob, out_hbm.at[pl.ds(base, CHUNK)], so)

            @pl.when(j % 2 == 0)
            def _():
                do(0)

            @pl.when(j % 2 == 1)
            def _():
                do(1)

        return 0

    lax.fori_loop(0, NITER, iter_body, 0)
    pltpu.make_async_copy(sets[0][1], out_hbm.at[pl.ds(0, CHUNK)], sets[0][3]).wait()
    pltpu.make_async_copy(sets[1][1], out_hbm.at[pl.ds(0, CHUNK)], sets[1][3]).wait()


@jax.jit
def kernel(values_l0, values_l1, values_l2, values_l3):
    packed = jnp.concatenate(
        [v.reshape(N_SAMPLES, -1) for v in
         (values_l0, values_l1, values_l2, values_l3)], axis=1)
    mesh = plsc.VectorSubcoreMesh(
        core_axis_name="c",
        subcore_axis_name="s",
        num_cores=NUM_CORES,
        num_subcores=NUM_SUBCORES,
    )
    scratch = [
        pltpu.VMEM((CHUNK, N_IN), jnp.float32),
        pltpu.VMEM((CHUNK, N_OUT), jnp.float32),
        pltpu.VMEM((CHUNK, N_IN), jnp.float32),
        pltpu.VMEM((CHUNK, N_OUT), jnp.float32),
    ] + [pltpu.SemaphoreType.DMA] * 4
    run = pl.kernel(
        _body,
        out_type=jax.ShapeDtypeStruct((N_SAMPLES, N_OUT), jnp.float32),
        mesh=mesh,
        scratch_types=scratch,
        compiler_params=pltpu.CompilerParams(use_tc_tiling_on_sc=True),
    )
    return run(packed)


# speedup vs baseline: 6.1684x; 1.3346x over previous
import math

import jax
import jax.numpy as jnp
from jax import lax
from jax.experimental import pallas as pl
from jax.experimental.pallas import tpu as pltpu
from jax.experimental.pallas import tpu_sc as plsc

N_SAMPLES = 20000
N_Q = 16
L_MAX = 3
N_OUT = 4 * N_Q * N_Q

NUM_CORES = 2
NUM_SUBCORES = 16
NUM_WORKERS = NUM_CORES * NUM_SUBCORES
CHUNK = 128
S_PAD = -(-N_SAMPLES // CHUNK) * CHUNK
NCHUNKS_TOTAL = S_PAD // CHUNK
NITER = -(-NCHUNKS_TOTAL // NUM_WORKERS)
NGROUPS = CHUNK // N_Q
NBLOCKS = 8


def _compute_block(tl, ob, l, qh):
    cg = 1.0 / math.sqrt(2 * l + 1)

    def group_body(g, _):
        s0 = g * N_Q

        def qpair_body(qp, _):
            q0 = qh * 8 + qp * 2
            acc0 = [None] * N_Q
            acc1 = [None] * N_Q
            for m in range(2 * l + 1):
                uq0 = tl[m, q0, pl.ds(s0, N_Q)] * cg
                uq1 = tl[m, q0 + 1, pl.ds(s0, N_Q)] * cg
                for p in range(N_Q):
                    up = tl[m, p, pl.ds(s0, N_Q)]
                    t0 = uq0 * up
                    t1 = uq1 * up
                    acc0[p] = t0 if m == 0 else acc0[p] + t0
                    acc1[p] = t1 if m == 0 else acc1[p] + t1
            r0 = qp * 32
            for p in range(N_Q):
                ob[r0 + p, pl.ds(s0, N_Q)] = acc0[p]
                ob[r0 + 16 + p, pl.ds(s0, N_Q)] = acc1[p]
            return 0

        lax.fori_loop(0, 8 // 2, qpair_body, 0)
        return 0

    lax.fori_loop(0, NGROUPS, group_body, 0)


def _body(v0, v1, v2, v3, out_hbm,
          a0, a1, a2, a3, b0, b1, b2, b3, oba, obb,
          sia, sib, soa, sob):
    wid = lax.axis_index("s") * NUM_CORES + lax.axis_index("c")
    in_hbm = (v0, v1, v2, v3)
    in_sets = ((a0, a1, a2, a3), (b0, b1, b2, b3))
    in_sems = (sia, sib)
    obufs = (oba, obb)
    osems = (soa, sob)

    def issue_in(par, base):
        for l in range(L_MAX + 1):
            pltpu.async_copy(
                in_hbm[l].at[:, :, pl.ds(base, CHUNK)], in_sets[par][l],
                in_sems[par])

    def wait_in(par):
        for l in range(L_MAX + 1):
            pltpu.make_async_copy(
                in_hbm[l].at[:, :, pl.ds(0, CHUNK)], in_sets[par][l],
                in_sems[par]).wait()

    def wait_out(i):
        pltpu.make_async_copy(
            obufs[i], out_hbm.at[i, :, pl.ds(0, CHUNK)], osems[i]).wait()

    issue_in(0, wid * CHUNK)

    def iter_body(j, _):
        c = wid + j * NUM_WORKERS

        @pl.when(c < NCHUNKS_TOTAL)
        def _():
            base = c * CHUNK

            def do(par):
                wait_in(par)

                @pl.when(c + NUM_WORKERS < NCHUNKS_TOTAL)
                def _():
                    issue_in(1 - par, base + NUM_WORKERS * CHUNK)

                for l in range(L_MAX + 1):
                    for qh in range(2):
                        blk = 2 * l + qh
                        ob = obufs[blk % 2]
                        if blk >= 2:
                            wait_out(blk % 2)
                        else:
                            @pl.when(j >= 1)
                            def _():
                                wait_out(blk % 2)
                        _compute_block(in_sets[par][l], ob, l, qh)
                        pltpu.async_copy(
                            ob, out_hbm.at[blk, :, pl.ds(base, CHUNK)],
                            osems[blk % 2])

            @pl.when(j % 2 == 0)
            def _():
                do(0)

            @pl.when(j % 2 == 1)
            def _():
                do(1)

        return 0

    lax.fori_loop(0, NITER, iter_body, 0)
    wait_out(0)
    wait_out(1)


@jax.jit
def kernel(values_l0, values_l1, values_l2, values_l3):
    pad = S_PAD - N_SAMPLES
    vts = [
        jnp.pad(jnp.transpose(v, (1, 2, 0)), ((0, 0), (0, 0), (0, pad)))
        for v in (values_l0, values_l1, values_l2, values_l3)
    ]
    mesh = plsc.VectorSubcoreMesh(
        core_axis_name="c",
        subcore_axis_name="s",
        num_cores=NUM_CORES,
        num_subcores=NUM_SUBCORES,
    )
    scratch = (
        [pltpu.VMEM((2 * l + 1, N_Q, CHUNK), jnp.float32)
         for l in range(L_MAX + 1)] * 2
        + [pltpu.VMEM((128, CHUNK), jnp.float32)] * 2
        + [pltpu.SemaphoreType.DMA] * 4
    )
    run = pl.kernel(
        _body,
        out_type=jax.ShapeDtypeStruct((NBLOCKS, 128, S_PAD), jnp.float32),
        mesh=mesh,
        scratch_types=scratch,
        compiler_params=pltpu.CompilerParams(use_tc_tiling_on_sc=True),
    )
    out_t = run(*vts)
    return jnp.transpose(out_t, (2, 0, 1)).reshape(S_PAD, N_OUT)[:N_SAMPLES]


# speedup vs baseline: 6.7273x; 1.0906x over previous
import math

import jax
import jax.numpy as jnp
from jax import lax
from jax.experimental import pallas as pl
from jax.experimental.pallas import tpu as pltpu
from jax.experimental.pallas import tpu_sc as plsc

N_SAMPLES = 20000
N_Q = 16
L_MAX = 3
N_OUT = 4 * N_Q * N_Q

NUM_CORES = 2
NUM_SUBCORES = 16
NUM_WORKERS = NUM_CORES * NUM_SUBCORES
CHUNK = 128
S_PAD = -(-N_SAMPLES // CHUNK) * CHUNK
NCHUNKS_TOTAL = S_PAD // CHUNK
NITER = -(-NCHUNKS_TOTAL // NUM_WORKERS)
NGROUPS = CHUNK // N_Q


def _compute_block(tl, ob, l):
    cg = 1.0 / math.sqrt(2 * l + 1)

    def group_body(g, _):
        s0 = g * N_Q
        for q0 in range(0, N_Q, 2):
            q1 = q0 + 1
            acc0 = {}
            acc1 = {}
            for m in range(2 * l + 1):
                uq0 = tl[m, q0, pl.ds(s0, N_Q)] * cg
                uq1 = tl[m, q1, pl.ds(s0, N_Q)] * cg
                for p in range(q0, N_Q):
                    up = tl[m, p, pl.ds(s0, N_Q)]
                    t0 = uq0 * up
                    acc0[p] = t0 if m == 0 else acc0[p] + t0
                    if p >= q1:
                        t1 = uq1 * up
                        acc1[p] = t1 if m == 0 else acc1[p] + t1
            for p in range(q0, N_Q):
                ob[q0 * N_Q + p, pl.ds(s0, N_Q)] = acc0[p]
                if p > q0:
                    ob[p * N_Q + q0, pl.ds(s0, N_Q)] = acc0[p]
                if p >= q1:
                    ob[q1 * N_Q + p, pl.ds(s0, N_Q)] = acc1[p]
                    if p > q1:
                        ob[p * N_Q + q1, pl.ds(s0, N_Q)] = acc1[p]
        return 0

    lax.fori_loop(0, NGROUPS, group_body, 0)


def _body(v0, v1, v2, v3, out_hbm, t0, t1, t2, t3, oba, obb, soa, sob):
    wid = lax.axis_index("s") * NUM_CORES + lax.axis_index("c")
    in_hbm = (v0, v1, v2, v3)
    tin = (t0, t1, t2, t3)
    obufs = (oba, obb)
    osems = (soa, sob)

    def wait_out(i):
        pltpu.make_async_copy(
            obufs[i], out_hbm.at[0, :, pl.ds(0, CHUNK)], osems[i]).wait()

    def iter_body(j, _):
        c = wid + j * NUM_WORKERS

        @pl.when(c < NCHUNKS_TOTAL)
        def _():
            base = c * CHUNK
            for l in range(L_MAX + 1):
                pltpu.sync_copy(
                    in_hbm[l].at[:, :, pl.ds(base, CHUNK)], tin[l])
            for l in range(L_MAX + 1):
                ob = obufs[l % 2]
                if l >= 2:
                    wait_out(l % 2)
                else:
                    @pl.when(j >= 1)
                    def _():
                        wait_out(l % 2)
                _compute_block(tin[l], ob, l)
                pltpu.async_copy(
                    ob, out_hbm.at[l, :, pl.ds(base, CHUNK)], osems[l % 2])

        return 0

    lax.fori_loop(0, NITER, iter_body, 0)
    wait_out(0)
    wait_out(1)


@jax.jit
def kernel(values_l0, values_l1, values_l2, values_l3):
    pad = S_PAD - N_SAMPLES
    vts = [
        jnp.pad(jnp.transpose(v, (1, 2, 0)), ((0, 0), (0, 0), (0, pad)))
        for v in (values_l0, values_l1, values_l2, values_l3)
    ]
    mesh = plsc.VectorSubcoreMesh(
        core_axis_name="c",
        subcore_axis_name="s",
        num_cores=NUM_CORES,
        num_subcores=NUM_SUBCORES,
    )
    scratch = (
        [pltpu.VMEM((2 * l + 1, N_Q, CHUNK), jnp.float32)
         for l in range(L_MAX + 1)]
        + [pltpu.VMEM((N_Q * N_Q, CHUNK), jnp.float32)] * 2
        + [pltpu.SemaphoreType.DMA] * 2
    )
    run = pl.kernel(
        _body,
        out_type=jax.ShapeDtypeStruct((L_MAX + 1, N_Q * N_Q, S_PAD),
                                      jnp.float32),
        mesh=mesh,
        scratch_types=scratch,
        compiler_params=pltpu.CompilerParams(use_tc_tiling_on_sc=True),
    )
    out_t = run(*vts)
    return jnp.transpose(out_t, (2, 0, 1)).reshape(S_PAD, N_OUT)[:N_SAMPLES]


# speedup vs baseline: 6.7371x; 1.0015x over previous
import math

import jax
import jax.numpy as jnp
from jax import lax
from jax.experimental import pallas as pl
from jax.experimental.pallas import tpu as pltpu
from jax.experimental.pallas import tpu_sc as plsc

N_SAMPLES = 20000
N_Q = 16
L_MAX = 3
N_OUT = 4 * N_Q * N_Q

NUM_CORES = 2
NUM_SUBCORES = 16
NUM_WORKERS = NUM_CORES * NUM_SUBCORES
CHUNK = 128
S_PAD = -(-N_SAMPLES // CHUNK) * CHUNK
NCHUNKS_TOTAL = S_PAD // CHUNK
NITER = -(-NCHUNKS_TOTAL // NUM_WORKERS)
NGROUPS = CHUNK // N_Q


def _compute_block(tl, ob, l):
    cg = 1.0 / math.sqrt(2 * l + 1)

    def group_body(g, _):
        s0 = g * N_Q
        for q0 in range(0, N_Q, 2):
            q1 = q0 + 1
            acc0 = {}
            acc1 = {}
            for m in range(2 * l + 1):
                up0 = tl[m, q0, pl.ds(s0, N_Q)]
                up1 = tl[m, q1, pl.ds(s0, N_Q)]
                uq0 = up0 * cg if l > 0 else up0
                uq1 = up1 * cg if l > 0 else up1
                for p in range(q0, N_Q):
                    if p == q0:
                        up = up0
                    elif p == q1:
                        up = up1
                    else:
                        up = tl[m, p, pl.ds(s0, N_Q)]
                    t0 = uq0 * up
                    acc0[p] = t0 if m == 0 else acc0[p] + t0
                    if p >= q1:
                        t1 = uq1 * up
                        acc1[p] = t1 if m == 0 else acc1[p] + t1
            for p in range(q0, N_Q):
                ob[q0 * N_Q + p, pl.ds(s0, N_Q)] = acc0[p]
                if p > q0:
                    ob[p * N_Q + q0, pl.ds(s0, N_Q)] = acc0[p]
                if p >= q1:
                    ob[q1 * N_Q + p, pl.ds(s0, N_Q)] = acc1[p]
                    if p > q1:
                        ob[p * N_Q + q1, pl.ds(s0, N_Q)] = acc1[p]
        return 0

    lax.fori_loop(0, NGROUPS, group_body, 0)


def _body(v0, v1, v2, v3, out_hbm, t0, t1, t2, t3, oba, obb, soa, sob):
    wid = lax.axis_index("s") * NUM_CORES + lax.axis_index("c")
    in_hbm = (v0, v1, v2, v3)
    tin = (t0, t1, t2, t3)
    obufs = (oba, obb)
    osems = (soa, sob)

    def wait_out(i):
        pltpu.make_async_copy(
            obufs[i], out_hbm.at[0, :, pl.ds(0, CHUNK)], osems[i]).wait()

    def iter_body(j, _):
        c = wid + j * NUM_WORKERS

        @pl.when(c < NCHUNKS_TOTAL)
        def _():
            base = c * CHUNK
            for l in range(L_MAX + 1):
                pltpu.sync_copy(
                    in_hbm[l].at[:, :, pl.ds(base, CHUNK)], tin[l])
            for l in range(L_MAX + 1):
                ob = obufs[l % 2]
                if l >= 2:
                    wait_out(l % 2)
                else:
                    @pl.when(j >= 1)
                    def _():
                        wait_out(l % 2)
                _compute_block(tin[l], ob, l)
                pltpu.async_copy(
                    ob, out_hbm.at[l, :, pl.ds(base, CHUNK)], osems[l % 2])

        return 0

    lax.fori_loop(0, NITER, iter_body, 0)
    wait_out(0)
    wait_out(1)


@jax.jit
def kernel(values_l0, values_l1, values_l2, values_l3):
    pad = S_PAD - N_SAMPLES
    vts = [
        jnp.pad(jnp.transpose(v, (1, 2, 0)), ((0, 0), (0, 0), (0, pad)))
        for v in (values_l0, values_l1, values_l2, values_l3)
    ]
    mesh = plsc.VectorSubcoreMesh(
        core_axis_name="c",
        subcore_axis_name="s",
        num_cores=NUM_CORES,
        num_subcores=NUM_SUBCORES,
    )
    scratch = (
        [pltpu.VMEM((2 * l + 1, N_Q, CHUNK), jnp.float32)
         for l in range(L_MAX + 1)]
        + [pltpu.VMEM((N_Q * N_Q, CHUNK), jnp.float32)] * 2
        + [pltpu.SemaphoreType.DMA] * 2
    )
    run = pl.kernel(
        _body,
        out_type=jax.ShapeDtypeStruct((L_MAX + 1, N_Q * N_Q, S_PAD),
                                      jnp.float32),
        mesh=mesh,
        scratch_types=scratch,
        compiler_params=pltpu.CompilerParams(use_tc_tiling_on_sc=True),
    )
    out_t = run(*vts)
    return jnp.transpose(out_t, (2, 0, 1)).reshape(S_PAD, N_OUT)[:N_SAMPLES]
